# baseline (speedup 1.0000x reference)
import functools
import math

import jax
import jax.numpy as jnp
from jax import lax
from jax.experimental import pallas as pl
from jax.experimental.pallas import tpu as pltpu
from jax.experimental.pallas import tpu_sc as plsc

IN_F = 2048
OUT_F = 2048
RANK = 64
SCALING = 16 / 64
NNZ = int(0.01 * IN_F * OUT_F)
W_WORDS = OUT_F * IN_F

N_WORKERS = 32
REGIONS_PER_WORKER = 2
N_REGIONS = N_WORKERS * REGIONS_PER_WORKER
REGION = W_WORDS // N_REGIONS
CHUNK = 1024
NK_PAD = 43008
LANES = 16


ROWS_PER_REGION = REGION // IN_F


def _sc_scatter_body(idx_hbm, val_hbm, bounds_hbm, w_hbm,
                     wbuf, idxbuf, valbuf, bbuf):
    wid = lax.axis_index("c") * 16 + lax.axis_index("s")
    pltpu.sync_copy(bounds_hbm, bbuf)
    bv = bbuf[pl.ds(pl.multiple_of(wid * LANES, LANES), LANES)]

    for p in range(REGIONS_PER_WORKER):
        r = wid * REGIONS_PER_WORKER + p
        base = r * REGION

        zero16 = jnp.zeros((LANES,), jnp.float32)

        def _zero(i, _):
            for u in range(IN_F // LANES):
                wbuf[i, pl.ds(u * LANES, LANES)] = zero16
            return 0

        lax.fori_loop(0, ROWS_PER_REGION, _zero, 0)

        s0 = bv[p]
        e0 = bv[p + 1]
        a0 = pl.multiple_of((s0 >> 3) << 3, 8)
        nch = (e0 - a0 + (CHUNK - 1)) >> 10

        def _chunk(t, _):
            off = pl.multiple_of(a0 + t * CHUNK, 8)
            pltpu.sync_copy(idx_hbm.at[pl.ds(off, CHUNK)], idxbuf)
            pltpu.sync_copy(val_hbm.at[pl.ds(off, CHUNK)], valbuf)

            def _scat(j, _):
                iv = idxbuf[pl.ds(j * LANES, LANES)]
                vv = valbuf[pl.ds(j * LANES, LANES)]
                m = (iv >= base) & (iv < base + REGION)
                lv = jnp.where(m, iv - base, 0)
                plsc.store_scatter(
                    wbuf, [lv >> 11, lv & (IN_F - 1)], vv, mask=m
                )
                return 0

            lax.fori_loop(0, CHUNK // LANES, _scat, 0)
            return 0

        lax.fori_loop(0, nch, _chunk, 0)

        row0 = pl.multiple_of(r * ROWS_PER_REGION, 8)
        pltpu.sync_copy(wbuf, w_hbm.at[pl.ds(row0, ROWS_PER_REGION)])


def _build_w_sp(idx_pad, val_pad, bounds):
    mesh = plsc.VectorSubcoreMesh(
        core_axis_name="c", subcore_axis_name="s", num_cores=2, num_subcores=16
    )
    k = pl.kernel(
        _sc_scatter_body,
        out_type=jax.ShapeDtypeStruct((OUT_F, IN_F), jnp.float32),
        mesh=mesh,
        scratch_types=[
            pltpu.VMEM((ROWS_PER_REGION, IN_F), jnp.float32),
            pltpu.VMEM((CHUNK,), jnp.int32),
            pltpu.VMEM((CHUNK,), jnp.float32),
            pltpu.VMEM((N_WORKERS * LANES,), jnp.int32),
        ],
        compiler_params=pltpu.CompilerParams(needs_layout_passes=False),
    )
    return k(idx_pad, val_pad, bounds)


BM = 512


def _mm_body(x_ref, wsp_ref, b_ref, a_ref, bias_ref, out_ref, wc_ref):
    @pl.when(pl.program_id(0) == 0)
    def _():
        lr = lax.dot_general(
            (b_ref[...] * SCALING).astype(jnp.bfloat16),
            a_ref[...].astype(jnp.bfloat16),
            (((1,), (0,)), ((), ())),
            preferred_element_type=jnp.float32,
        )
        wc_ref[...] = (wsp_ref[...] + lr).astype(jnp.bfloat16)

    acc = lax.dot_general(
        x_ref[...],
        wc_ref[...],
        (((1,), (1,)), ((), ())),
        preferred_element_type=jnp.float32,
    )
    out_ref[...] = acc + bias_ref[...]


def _fused_matmul(x2d, w_sp, lora_B, lora_A, bias2d):
    m_blocks = x2d.shape[0] // BM
    return pl.pallas_call(
        _mm_body,
        grid=(m_blocks,),
        in_specs=[
            pl.BlockSpec((BM, IN_F), lambda m: (m, 0)),
            pl.BlockSpec((OUT_F, IN_F), lambda m: (0, 0)),
            pl.BlockSpec((OUT_F, RANK), lambda m: (0, 0)),
            pl.BlockSpec((RANK, IN_F), lambda m: (0, 0)),
            pl.BlockSpec((1, OUT_F), lambda m: (0, 0)),
        ],
        out_specs=pl.BlockSpec((BM, OUT_F), lambda m: (m, 0)),
        out_shape=jax.ShapeDtypeStruct((x2d.shape[0], OUT_F), jnp.float32),
        scratch_shapes=[pltpu.VMEM((OUT_F, IN_F), jnp.bfloat16)],
    )(x2d, w_sp, lora_B, lora_A, bias2d)


def kernel(x, lora_B, lora_A, sparse_value, sparse_index, bias):
    b, s, _ = x.shape
    idx = sparse_index.astype(jnp.int32)
    pad = NK_PAD - NNZ
    idx_pad = jnp.concatenate([idx, jnp.full((pad,), -1, jnp.int32)])
    val_pad = jnp.concatenate([sparse_value, jnp.zeros((pad,), jnp.float32)])
    boundaries = jnp.arange(N_REGIONS + 1, dtype=jnp.int32) * REGION
    bounds = jnp.sum(
        (idx[None, :] < boundaries[:, None]).astype(jnp.int32), axis=1
    )
    row_idx = (
        jnp.arange(N_WORKERS)[:, None] * REGIONS_PER_WORKER
        + jnp.arange(REGIONS_PER_WORKER + 1)[None, :]
    )
    bounds_rows = jnp.pad(
        bounds[row_idx], ((0, 0), (0, LANES - REGIONS_PER_WORKER - 1))
    ).reshape(-1)
    w_sp = _build_w_sp(idx_pad, val_pad, bounds_rows)

    x2d = x.reshape(b * s, IN_F).astype(jnp.bfloat16)
    out2d = _fused_matmul(x2d, w_sp, lora_B, lora_A, bias.reshape(1, OUT_F))
    return out2d.reshape(b, s, OUT_F)


# speedup vs baseline: 1.0992x; 1.0992x over previous
import functools
import math

import jax
import jax.numpy as jnp
from jax import lax
from jax.experimental import pallas as pl
from jax.experimental.pallas import tpu as pltpu
from jax.experimental.pallas import tpu_sc as plsc

IN_F = 2048
OUT_F = 2048
RANK = 64
SCALING = 16 / 64
NNZ = int(0.01 * IN_F * OUT_F)
W_WORDS = OUT_F * IN_F

N_WORKERS = 32
REGIONS_PER_WORKER = 2
N_REGIONS = N_WORKERS * REGIONS_PER_WORKER
REGION = W_WORDS // N_REGIONS
CHUNK = 1024
NK_PAD = 43008
LANES = 16


ROWS_PER_REGION = REGION // IN_F


def _sc_scatter_body(idx_hbm, val_hbm, bounds_hbm, w_hbm,
                     wbuf, idxbuf, valbuf, bbuf):
    wid = lax.axis_index("c") * 16 + lax.axis_index("s")
    pltpu.sync_copy(bounds_hbm, bbuf)
    bv = bbuf[pl.ds(pl.multiple_of(wid * LANES, LANES), LANES)]

    for p in range(REGIONS_PER_WORKER):
        r = wid * REGIONS_PER_WORKER + p
        base = r * REGION

        zero16 = jnp.zeros((LANES,), jnp.float32)

        def _zero(i, _):
            for u in range(IN_F // LANES):
                wbuf[i, pl.ds(u * LANES, LANES)] = zero16
            return 0

        lax.fori_loop(0, ROWS_PER_REGION, _zero, 0)

        s0 = bv[p]
        e0 = bv[p + 1]
        a0 = pl.multiple_of((s0 >> 3) << 3, 8)
        nch = (e0 - a0 + (CHUNK - 1)) >> 10

        def _chunk(t, _):
            off = pl.multiple_of(a0 + t * CHUNK, 8)
            pltpu.sync_copy(idx_hbm.at[pl.ds(off, CHUNK)], idxbuf)
            pltpu.sync_copy(val_hbm.at[pl.ds(off, CHUNK)], valbuf)

            def _scat(j, _):
                iv = idxbuf[pl.ds(j * LANES, LANES)]
                vv = valbuf[pl.ds(j * LANES, LANES)]
                m = (iv >= base) & (iv < base + REGION)
                lv = jnp.where(m, iv - base, 0)
                plsc.store_scatter(
                    wbuf, [lv >> 11, lv & (IN_F - 1)], vv, mask=m
                )
                return 0

            lax.fori_loop(0, CHUNK // LANES, _scat, 0)
            return 0

        lax.fori_loop(0, nch, _chunk, 0)

        row0 = pl.multiple_of(r * ROWS_PER_REGION, 8)
        pltpu.sync_copy(wbuf, w_hbm.at[pl.ds(row0, ROWS_PER_REGION)])


def _build_w_sp(idx_pad, val_pad, bounds):
    mesh = plsc.VectorSubcoreMesh(
        core_axis_name="c", subcore_axis_name="s", num_cores=2, num_subcores=16
    )
    k = pl.kernel(
        _sc_scatter_body,
        out_type=jax.ShapeDtypeStruct((OUT_F, IN_F), jnp.float32),
        mesh=mesh,
        scratch_types=[
            pltpu.VMEM((ROWS_PER_REGION, IN_F), jnp.float32),
            pltpu.VMEM((CHUNK,), jnp.int32),
            pltpu.VMEM((CHUNK,), jnp.float32),
            pltpu.VMEM((N_WORKERS * LANES,), jnp.int32),
        ],
        compiler_params=pltpu.CompilerParams(needs_layout_passes=False),
    )
    return k(idx_pad, val_pad, bounds)


BM = 512


def _mm_body(x_ref, wsp_ref, b_ref, a_ref, bias_ref, out_ref, wc_ref):
    @pl.when(pl.program_id(0) == 0)
    def _():
        lr = lax.dot_general(
            (b_ref[...] * SCALING).astype(jnp.bfloat16),
            a_ref[...].astype(jnp.bfloat16),
            (((1,), (0,)), ((), ())),
            preferred_element_type=jnp.float32,
        )
        wc_ref[...] = (wsp_ref[...] + lr).astype(jnp.bfloat16)

    acc = lax.dot_general(
        x_ref[...].astype(jnp.bfloat16),
        wc_ref[...],
        (((1,), (1,)), ((), ())),
        preferred_element_type=jnp.float32,
    )
    out_ref[...] = acc + bias_ref[...]


def _fused_matmul(x2d, w_sp, lora_B, lora_A, bias2d):
    m_blocks = x2d.shape[0] // BM
    return pl.pallas_call(
        _mm_body,
        grid=(m_blocks,),
        in_specs=[
            pl.BlockSpec((BM, IN_F), lambda m: (m, 0)),
            pl.BlockSpec((OUT_F, IN_F), lambda m: (0, 0)),
            pl.BlockSpec((OUT_F, RANK), lambda m: (0, 0)),
            pl.BlockSpec((RANK, IN_F), lambda m: (0, 0)),
            pl.BlockSpec((1, OUT_F), lambda m: (0, 0)),
        ],
        out_specs=pl.BlockSpec((BM, OUT_F), lambda m: (m, 0)),
        out_shape=jax.ShapeDtypeStruct((x2d.shape[0], OUT_F), jnp.float32),
        scratch_shapes=[pltpu.VMEM((OUT_F, IN_F), jnp.bfloat16)],
    )(x2d, w_sp, lora_B, lora_A, bias2d)


def kernel(x, lora_B, lora_A, sparse_value, sparse_index, bias):
    b, s, _ = x.shape
    idx = sparse_index.astype(jnp.int32)
    pad = NK_PAD - NNZ
    idx_pad = jnp.concatenate([idx, jnp.full((pad,), -1, jnp.int32)])
    val_pad = jnp.concatenate([sparse_value, jnp.zeros((pad,), jnp.float32)])
    boundaries = jnp.arange(N_REGIONS + 1, dtype=jnp.int32) * REGION
    bounds = jnp.sum(
        (idx[None, :] < boundaries[:, None]).astype(jnp.int32), axis=1
    )
    row_idx = (
        jnp.arange(N_WORKERS)[:, None] * REGIONS_PER_WORKER
        + jnp.arange(REGIONS_PER_WORKER + 1)[None, :]
    )
    bounds_rows = jnp.pad(
        bounds[row_idx], ((0, 0), (0, LANES - REGIONS_PER_WORKER - 1))
    ).reshape(-1)
    w_sp = _build_w_sp(idx_pad, val_pad, bounds_rows)

    x2d = x.reshape(b * s, IN_F)
    out2d = _fused_matmul(x2d, w_sp, lora_B, lora_A, bias.reshape(1, OUT_F))
    return out2d.reshape(b, s, OUT_F)
